# baseline (speedup 1.0000x reference)
import functools

import jax
import jax.numpy as jnp
from jax import lax
from jax.experimental import pallas as pl
from jax.experimental.pallas import tpu as pltpu, tpu_sc as plsc

BATCH = 4
SEQ = 2048
HIDDEN = 768
VOCAB = 100000
TYPE_VOCAB = 2
EPS = 1e-12

TOK = BATCH * SEQ
NC, NS, LANES = 2, 16, 16
NW = NC * NS
TPW = TOK // NW
CHUNK = 64
NCHUNK = TPW // CHUNK
KH = HIDDEN // LANES

_mesh = plsc.VectorSubcoreMesh(core_axis_name="c", subcore_axis_name="s")


@functools.partial(
    pl.kernel,
    out_type=jax.ShapeDtypeStruct((TOK, HIDDEN), jnp.float32),
    mesh=_mesh,
    compiler_params=pltpu.CompilerParams(needs_layout_passes=False),
    scratch_types=[
        pltpu.VMEM((CHUNK,), jnp.int32),
        pltpu.VMEM((CHUNK, LANES), jnp.float32),
        pltpu.VMEM((CHUNK, HIDDEN), jnp.float32),
        pltpu.VMEM((CHUNK, HIDDEN), jnp.float32),
        pltpu.VMEM((TYPE_VOCAB, HIDDEN), jnp.float32),
        pltpu.VMEM((HIDDEN,), jnp.float32),
        pltpu.VMEM((HIDDEN,), jnp.float32),
        pltpu.VMEM((HIDDEN,), jnp.float32),
        pltpu.VMEM((HIDDEN,), jnp.float32),
        pltpu.SemaphoreType.DMA,
    ],
)
def _embed_ln(ids_hbm, ttb_hbm, word_hbm, pos_hbm, type_hbm, gamma_hbm,
              beta_hbm, out_hbm, idx_v, ttb_v, w_v, pos_v, type_v,
              t0_v, td_v, gamma_v, beta_v, sem):
    wid = lax.axis_index("s") * NC + lax.axis_index("c")
    base = wid * TPW

    pltpu.sync_copy(type_hbm, type_v)
    pltpu.sync_copy(gamma_hbm, gamma_v)
    pltpu.sync_copy(beta_hbm, beta_v)
    for k in range(KH):
        r0 = type_v[0, pl.ds(LANES * k, LANES)]
        r1 = type_v[1, pl.ds(LANES * k, LANES)]
        t0_v[pl.ds(LANES * k, LANES)] = r0
        td_v[pl.ds(LANES * k, LANES)] = r1 - r0

    def tok_body(t, carry):
        ttf = ttb_v[t, :]
        acc_s = jnp.zeros((LANES,), jnp.float32)
        acc_q = jnp.zeros((LANES,), jnp.float32)
        for k in range(KH):
            w = w_v[t, pl.ds(LANES * k, LANES)]
            p = pos_v[t, pl.ds(LANES * k, LANES)]
            t0 = t0_v[pl.ds(LANES * k, LANES)]
            td = td_v[pl.ds(LANES * k, LANES)]
            v = w + p + (t0 + ttf * td)
            w_v[t, pl.ds(LANES * k, LANES)] = v
            acc_s = acc_s + v
            acc_q = acc_q + v * v
        s = jnp.sum(acc_s)
        q = jnp.sum(acc_q)
        mean = s * (1.0 / HIDDEN)
        var = q * (1.0 / HIDDEN) - mean * mean
        x = jnp.full((LANES,), var + EPS, jnp.float32)
        mean_b = jnp.full((LANES,), mean, jnp.float32)
        i = lax.bitcast_convert_type(x, jnp.int32)
        magic = jnp.full((LANES,), 0x5F3759DF, jnp.int32)
        one = jnp.full((LANES,), 1, jnp.int32)
        y = lax.bitcast_convert_type(magic - lax.shift_right_arithmetic(i, one),
                                     jnp.float32)
        for _ in range(3):
            y = y * (1.5 - 0.5 * x * y * y)
        for k in range(KH):
            g = gamma_v[pl.ds(LANES * k, LANES)]
            b = beta_v[pl.ds(LANES * k, LANES)]
            v = w_v[t, pl.ds(LANES * k, LANES)]
            w_v[t, pl.ds(LANES * k, LANES)] = (v - mean_b) * y * g + b
        return carry

    for c in range(NCHUNK):
        tbase = base + c * CHUNK
        pbase = tbase % SEQ
        pltpu.sync_copy(ids_hbm.at[pl.ds(tbase, CHUNK)], idx_v)
        pltpu.sync_copy(ttb_hbm.at[pl.ds(tbase, CHUNK)], ttb_v)
        pltpu.async_copy(word_hbm.at[idx_v], w_v, sem).wait()
        pltpu.sync_copy(pos_hbm.at[pl.ds(pbase, CHUNK)], pos_v)
        lax.fori_loop(0, CHUNK, tok_body, 0)
        pltpu.sync_copy(w_v, out_hbm.at[pl.ds(tbase, CHUNK)])


def kernel(input_ids, token_type_ids, word_embeddings, position_embeddings,
           token_type_embeddings, ln_gamma, ln_beta):
    ids = input_ids.reshape(TOK).astype(jnp.int32)
    ttb = jnp.broadcast_to(
        token_type_ids.reshape(TOK, 1).astype(jnp.float32), (TOK, LANES))
    out = _embed_ln(ids, ttb, word_embeddings, position_embeddings,
                    token_type_embeddings, ln_gamma, ln_beta)
    return out.reshape(BATCH, SEQ, HIDDEN)


# speedup vs baseline: 1.8777x; 1.8777x over previous
import functools

import jax
import jax.numpy as jnp
from jax import lax
from jax.experimental import pallas as pl
from jax.experimental.pallas import tpu as pltpu, tpu_sc as plsc

BATCH = 4
SEQ = 2048
HIDDEN = 768
VOCAB = 100000
TYPE_VOCAB = 2
EPS = 1e-12

TOK = BATCH * SEQ
NC, NS, LANES = 2, 16, 16
NW = NC * NS
TPW = TOK // NW
CHUNK = 64
NCHUNK = TPW // CHUNK
KH = HIDDEN // LANES

_mesh = plsc.VectorSubcoreMesh(core_axis_name="c", subcore_axis_name="s")


@functools.partial(
    pl.kernel,
    out_type=jax.ShapeDtypeStruct((TOK, HIDDEN), jnp.float32),
    mesh=_mesh,
    compiler_params=pltpu.CompilerParams(needs_layout_passes=False),
    scratch_types=[
        pltpu.VMEM((CHUNK,), jnp.int32),
        pltpu.VMEM((CHUNK, LANES), jnp.float32),
        pltpu.VMEM((CHUNK, HIDDEN), jnp.float32),
        pltpu.VMEM((CHUNK, HIDDEN), jnp.float32),
        pltpu.VMEM((TYPE_VOCAB, HIDDEN), jnp.float32),
        pltpu.VMEM((HIDDEN,), jnp.float32),
        pltpu.VMEM((HIDDEN,), jnp.float32),
        pltpu.VMEM((HIDDEN,), jnp.float32),
        pltpu.VMEM((HIDDEN,), jnp.float32),
        pltpu.SemaphoreType.DMA,
    ],
)
def _embed_ln(ids_hbm, ttb_hbm, word_hbm, pos_hbm, type_hbm, gamma_hbm,
              beta_hbm, out_hbm, idx_v, ttb_v, w_v, pos_v, type_v,
              t0_v, td_v, gamma_v, beta_v, sem):
    wid = lax.axis_index("s") * NC + lax.axis_index("c")
    base = wid * TPW

    pltpu.sync_copy(type_hbm, type_v)
    pltpu.sync_copy(gamma_hbm, gamma_v)
    pltpu.sync_copy(beta_hbm, beta_v)
    for k in range(KH):
        r0 = type_v[0, pl.ds(LANES * k, LANES)]
        r1 = type_v[1, pl.ds(LANES * k, LANES)]
        t0_v[pl.ds(LANES * k, LANES)] = r0
        td_v[pl.ds(LANES * k, LANES)] = r1 - r0

    def tok_body(t):
        ttf = ttb_v[t, :]
        acc_s = jnp.zeros((LANES,), jnp.float32)
        acc_q = jnp.zeros((LANES,), jnp.float32)
        for k in range(KH):
            w = w_v[t, pl.ds(LANES * k, LANES)]
            p = pos_v[t, pl.ds(LANES * k, LANES)]
            t0 = t0_v[pl.ds(LANES * k, LANES)]
            td = td_v[pl.ds(LANES * k, LANES)]
            v = w + p + (t0 + ttf * td)
            w_v[t, pl.ds(LANES * k, LANES)] = v
            acc_s = acc_s + v
            acc_q = acc_q + v * v
        s = jnp.sum(acc_s)
        q = jnp.sum(acc_q)
        mean = s * (1.0 / HIDDEN)
        var = q * (1.0 / HIDDEN) - mean * mean
        x = jnp.full((LANES,), var + EPS, jnp.float32)
        mean_b = jnp.full((LANES,), mean, jnp.float32)
        i = lax.bitcast_convert_type(x, jnp.int32)
        magic = jnp.full((LANES,), 0x5F3759DF, jnp.int32)
        one = jnp.full((LANES,), 1, jnp.int32)
        y = lax.bitcast_convert_type(magic - lax.shift_right_arithmetic(i, one),
                                     jnp.float32)
        for _ in range(3):
            y = y * (1.5 - 0.5 * x * y * y)
        for k in range(KH):
            g = gamma_v[pl.ds(LANES * k, LANES)]
            b = beta_v[pl.ds(LANES * k, LANES)]
            v = w_v[t, pl.ds(LANES * k, LANES)]
            w_v[t, pl.ds(LANES * k, LANES)] = (v - mean_b) * y * g + b

    def chunk_body(c, carry):
        tbase = pl.multiple_of(base + c * CHUNK, CHUNK)
        pbase = pl.multiple_of(lax.rem(base + c * CHUNK, SEQ), CHUNK)
        pltpu.sync_copy(ids_hbm.at[pl.ds(tbase, CHUNK)], idx_v)
        pltpu.sync_copy(ttb_hbm.at[pl.ds(tbase, CHUNK)], ttb_v)
        pltpu.async_copy(word_hbm.at[idx_v], w_v, sem).wait()
        pltpu.sync_copy(pos_hbm.at[pl.ds(pbase, CHUNK)], pos_v)
        plsc.parallel_loop(0, CHUNK, unroll=1)(tok_body)
        pltpu.sync_copy(w_v, out_hbm.at[pl.ds(tbase, CHUNK)])
        return carry

    lax.fori_loop(0, NCHUNK, chunk_body, 0)


def kernel(input_ids, token_type_ids, word_embeddings, position_embeddings,
           token_type_embeddings, ln_gamma, ln_beta):
    ids = input_ids.reshape(TOK).astype(jnp.int32)
    ttb = jnp.broadcast_to(
        token_type_ids.reshape(TOK, 1).astype(jnp.float32), (TOK, LANES))
    out = _embed_ln(ids, ttb, word_embeddings, position_embeddings,
                    token_type_embeddings, ln_gamma, ln_beta)
    return out.reshape(BATCH, SEQ, HIDDEN)
